# Initial kernel scaffold; baseline (speedup 1.0000x reference)
import jax
import jax.numpy as jnp
from jax.experimental import pallas as pl


def kernel(static_mask, code, numerical_value, time_delta_days, numerical_value_mask, mask, code_table, date_W1, date_b1, date_W2, date_b2, val_W1, val_b1, val_W2, val_b2, ts_token, code_prefix, val_prefix):
    raise NotImplementedError("write your pallas kernel here")


# speedup vs baseline: 2.1523x; 2.1523x over previous
import functools

import jax
import jax.numpy as jnp
from jax import lax
from jax.experimental import pallas as pl
from jax.experimental.pallas import tpu as pltpu
from jax.experimental.pallas import tpu_sc as plsc

TOKEN_DIM = 1024
N_ROWS = 8192


_NC = 2
_NS = 16
_NW = _NC * _NS
_BPW = N_ROWS // _NW
_CH = 32
_NCHUNK = _BPW // _CH


def _sc_gather_build():
    mesh = plsc.VectorSubcoreMesh(core_axis_name="c", subcore_axis_name="s")

    @functools.partial(
        pl.kernel,
        mesh=mesh,
        out_type=jax.ShapeDtypeStruct((N_ROWS, TOKEN_DIM), jnp.float32),
        scratch_types=[
            pltpu.VMEM((_BPW,), jnp.int32),
            pltpu.VMEM((_CH, TOKEN_DIM), jnp.float32),
            pltpu.VMEM((_CH, TOKEN_DIM), jnp.float32),
            pltpu.SemaphoreType.DMA,
            pltpu.SemaphoreType.DMA,
        ],
    )
    def gather_kernel(idx_hbm, table_hbm, out_hbm, idx_v, buf0, buf1, sem0, sem1):
        wid = lax.axis_index("s") * _NC + lax.axis_index("c")
        base = wid * _BPW
        pltpu.sync_copy(idx_hbm.at[pl.ds(base, _BPW)], idx_v)

        bufs = (buf0, buf1)
        sems = (sem0, sem1)

        def start(c):
            return pltpu.async_copy(
                table_hbm.at[idx_v.at[pl.ds(c * _CH, _CH)]],
                bufs[c % 2],
                sems[c % 2],
            )

        cur = start(0)
        for c in range(_NCHUNK):
            nxt = start(c + 1) if c + 1 < _NCHUNK else None
            cur.wait()
            pltpu.sync_copy(bufs[c % 2], out_hbm.at[pl.ds(base + c * _CH, _CH)])
            cur = nxt

    return gather_kernel


_SC_GATHER_CACHE = []


def _sc_gather(idx, table):
    if not _SC_GATHER_CACHE:
        _SC_GATHER_CACHE.append(_sc_gather_build())
    return _SC_GATHER_CACHE[0](idx, table)


_BR = 256
_GRID = N_ROWS // _BR


def _assemble_body(td, nv, sm, nvm, ce,
                   dW1, db1, dW2, db2,
                   vW1, vb1, vW2, vb2,
                   tst, cpf, vpf, out):
    t = td[...]
    h = jnp.tanh(t * dW1[...] + db1[...])
    temb = jnp.dot(h, dW2[...],
                   preferred_element_type=jnp.float32) + db2[...]
    tvalid = (t != 0.0) & (sm[...] != 0.0)
    out[:, 0:TOKEN_DIM] = jnp.where(tvalid, temb, tst[...])

    out[:, TOKEN_DIM:2 * TOKEN_DIM] = jnp.broadcast_to(cpf[...], (_BR, TOKEN_DIM))
    out[:, 2 * TOKEN_DIM:3 * TOKEN_DIM] = ce[...]
    out[:, 3 * TOKEN_DIM:4 * TOKEN_DIM] = jnp.broadcast_to(vpf[...], (_BR, TOKEN_DIM))

    v = nv[...]
    hv = jnp.tanh(v * vW1[...] + vb1[...])
    vemb = jnp.dot(hv, vW2[...],
                   preferred_element_type=jnp.float32) + vb2[...]
    vvalid = nvm[...] != 0.0
    out[:, 4 * TOKEN_DIM:5 * TOKEN_DIM] = jnp.where(vvalid, vemb, vpf[...])


def _row_spec():
    return pl.BlockSpec((_BR, 1), lambda i: (i, 0))


def _full_spec(shape):
    return pl.BlockSpec(shape, lambda i: tuple(0 for _ in shape))


def _tc_assemble(td, nv, sm, nvm, code_emb,
                 dW1, db1, dW2, db2, vW1, vb1, vW2, vb2,
                 tst, cpf, vpf):
    return pl.pallas_call(
        _assemble_body,
        grid=(_GRID,),
        in_specs=[
            _row_spec(), _row_spec(), _row_spec(), _row_spec(),
            pl.BlockSpec((_BR, TOKEN_DIM), lambda i: (i, 0)),
            _full_spec((1, 32)), _full_spec((1, 32)),
            _full_spec((32, TOKEN_DIM)), _full_spec((1, TOKEN_DIM)),
            _full_spec((1, 32)), _full_spec((1, 32)),
            _full_spec((32, TOKEN_DIM)), _full_spec((1, TOKEN_DIM)),
            _full_spec((1, TOKEN_DIM)), _full_spec((1, TOKEN_DIM)),
            _full_spec((1, TOKEN_DIM)),
        ],
        out_specs=pl.BlockSpec((_BR, 5 * TOKEN_DIM), lambda i: (i, 0)),
        out_shape=jax.ShapeDtypeStruct((N_ROWS, 5 * TOKEN_DIM), jnp.float32),
    )(td, nv, sm, nvm, code_emb,
      dW1, db1, dW2, db2, vW1, vb1, vW2, vb2, tst, cpf, vpf)


def kernel(static_mask, code, numerical_value, time_delta_days,
           numerical_value_mask, mask, code_table,
           date_W1, date_b1, date_W2, date_b2,
           val_W1, val_b1, val_W2, val_b2,
           ts_token, code_prefix, val_prefix):
    n = code.shape[0]
    code_emb = _sc_gather(code.astype(jnp.int32), code_table)

    out = _tc_assemble(
        time_delta_days.reshape(n, 1),
        numerical_value.reshape(n, 1),
        static_mask.astype(jnp.float32).reshape(n, 1),
        numerical_value_mask.astype(jnp.float32).reshape(n, 1),
        code_emb,
        date_W1, date_b1.reshape(1, 32), date_W2, date_b2.reshape(1, TOKEN_DIM),
        val_W1, val_b1.reshape(1, 32), val_W2, val_b2.reshape(1, TOKEN_DIM),
        ts_token.reshape(1, TOKEN_DIM),
        code_prefix.reshape(1, TOKEN_DIM),
        val_prefix.reshape(1, TOKEN_DIM),
    )
    return out
